# Initial kernel scaffold; baseline (speedup 1.0000x reference)
import jax
import jax.numpy as jnp
from jax.experimental import pallas as pl


def kernel(x, gamma, beta, shared_w1, shared_w2, experts_w1, experts_w2, router_w):
    raise NotImplementedError("write your pallas kernel here")


# speedup vs baseline: 1.1502x; 1.1502x over previous
import functools

import jax
import jax.numpy as jnp
from jax.experimental import pallas as pl
from jax.experimental.pallas import tpu as pltpu

D_MODEL = 1024
EXPERT_DIM = 2048
N_EXPERTS = 8
T = 2 * 2048
BT = 256
NB = (T + N_EXPERTS * BT) // BT
T_PAD = NB * BT


def _quant_body(w_ref, o_ref):
    w = w_ref[...]
    scale = jnp.clip(jnp.mean(jnp.abs(w), axis=1, keepdims=True), 1e-05, None)
    wq = jnp.clip(jnp.round(w / scale), -1.0, 1.0)
    o_ref[...] = wq * scale


def _quantize(w):
    r, c = w.shape
    rb = min(r, 512)
    return pl.pallas_call(
        _quant_body,
        grid=(r // rb,),
        in_specs=[pl.BlockSpec((rb, c), lambda i: (i, 0))],
        out_specs=pl.BlockSpec((rb, c), lambda i: (i, 0)),
        out_shape=jax.ShapeDtypeStruct((r, c), jnp.float32),
    )(w)


def _ln_router_body(x_ref, g_ref, b_ref, rw_ref, h_ref, p_ref, i_ref):
    x = x_ref[...]
    mu = jnp.mean(x, axis=1, keepdims=True)
    var = jnp.mean(jnp.square(x - mu), axis=1, keepdims=True)
    h = (x - mu) / jnp.sqrt(var + 1e-05) * g_ref[...] + b_ref[...]
    h_ref[...] = h
    logits = jax.lax.dot_general(h, rw_ref[...], (((1,), (1,)), ((), ())),
                                 preferred_element_type=jnp.float32)
    m = jnp.max(logits, axis=1, keepdims=True)
    e = jnp.exp(logits - m)
    p = e / jnp.sum(e, axis=1, keepdims=True)
    pm = jnp.max(p, axis=1, keepdims=True)
    lane = jax.lax.broadcasted_iota(jnp.int32, p.shape, 1)
    idx = jnp.min(jnp.where(p == pm, lane, N_EXPERTS), axis=1, keepdims=True)
    p_ref[...] = pm
    i_ref[...] = idx


def _ln_router(x2d, gamma, beta, router_w):
    return pl.pallas_call(
        _ln_router_body,
        grid=(T // BT,),
        in_specs=[
            pl.BlockSpec((BT, D_MODEL), lambda i: (i, 0)),
            pl.BlockSpec((1, D_MODEL), lambda i: (0, 0)),
            pl.BlockSpec((1, D_MODEL), lambda i: (0, 0)),
            pl.BlockSpec((N_EXPERTS, D_MODEL), lambda i: (0, 0)),
        ],
        out_specs=[
            pl.BlockSpec((BT, D_MODEL), lambda i: (i, 0)),
            pl.BlockSpec((BT, 1), lambda i: (i, 0)),
            pl.BlockSpec((BT, 1), lambda i: (i, 0)),
        ],
        out_shape=[
            jax.ShapeDtypeStruct((T, D_MODEL), jnp.float32),
            jax.ShapeDtypeStruct((T, 1), jnp.float32),
            jax.ShapeDtypeStruct((T, 1), jnp.int32),
        ],
    )(x2d, gamma.reshape(1, -1), beta.reshape(1, -1), router_w)


def _shared_ffn_body(h_ref, w1_ref, w2_ref, o_ref):
    a = jax.lax.dot_general(h_ref[...], w1_ref[...], (((1,), (1,)), ((), ())),
                            preferred_element_type=jnp.float32)
    a = a * jax.lax.logistic(a)
    o_ref[...] = jax.lax.dot_general(a, w2_ref[...], (((1,), (1,)), ((), ())),
                                     preferred_element_type=jnp.float32)


def _shared_ffn(h, w1q, w2q):
    return pl.pallas_call(
        _shared_ffn_body,
        grid=(T // BT,),
        in_specs=[
            pl.BlockSpec((BT, D_MODEL), lambda i: (i, 0)),
            pl.BlockSpec((EXPERT_DIM, D_MODEL), lambda i: (0, 0)),
            pl.BlockSpec((D_MODEL, EXPERT_DIM), lambda i: (0, 0)),
        ],
        out_specs=pl.BlockSpec((BT, D_MODEL), lambda i: (i, 0)),
        out_shape=jax.ShapeDtypeStruct((T, D_MODEL), jnp.float32),
    )(h, w1q, w2q)


def _moe_ffn_body(be_ref, valid_ref, hs_ref, w1_ref, w2_ref, ps_ref, o_ref):
    i = pl.program_id(0)

    @pl.when(valid_ref[i] != 0)
    def _():
        a = jax.lax.dot_general(hs_ref[...], w1_ref[0], (((1,), (1,)), ((), ())),
                                preferred_element_type=jnp.float32)
        a = a * jax.lax.logistic(a)
        o = jax.lax.dot_general(a, w2_ref[0], (((1,), (1,)), ((), ())),
                                preferred_element_type=jnp.float32)
        o_ref[...] = o * ps_ref[...]


def _moe_ffn(h_sorted, ew1q, ew2q, probs_sorted, block_e, block_valid):
    grid_spec = pltpu.PrefetchScalarGridSpec(
        num_scalar_prefetch=2,
        grid=(NB,),
        in_specs=[
            pl.BlockSpec((BT, D_MODEL), lambda i, be, v: (i, 0)),
            pl.BlockSpec((1, EXPERT_DIM, D_MODEL), lambda i, be, v: (be[i], 0, 0)),
            pl.BlockSpec((1, D_MODEL, EXPERT_DIM), lambda i, be, v: (be[i], 0, 0)),
            pl.BlockSpec((BT, 1), lambda i, be, v: (i, 0)),
        ],
        out_specs=pl.BlockSpec((BT, D_MODEL), lambda i, be, v: (i, 0)),
    )
    return pl.pallas_call(
        _moe_ffn_body,
        grid_spec=grid_spec,
        out_shape=jax.ShapeDtypeStruct((T_PAD, D_MODEL), jnp.float32),
    )(block_e, block_valid, h_sorted, ew1q, ew2q, probs_sorted)


def kernel(x, gamma, beta, shared_w1, shared_w2, experts_w1, experts_w2, router_w):
    x2d = x.reshape(T, D_MODEL)

    sw1q = _quantize(shared_w1)
    sw2q = _quantize(shared_w2)
    ew1q = _quantize(experts_w1.reshape(N_EXPERTS * EXPERT_DIM, D_MODEL)).reshape(
        N_EXPERTS, EXPERT_DIM, D_MODEL)
    ew2q = _quantize(experts_w2.reshape(N_EXPERTS * D_MODEL, EXPERT_DIM)).reshape(
        N_EXPERTS, D_MODEL, EXPERT_DIM)

    h, topk_prob, topk_idx = _ln_router(x2d, gamma, beta, router_w)
    topk_prob = topk_prob[:, 0]
    topk_idx = topk_idx[:, 0]

    sort_idx = jnp.argsort(topk_idx)
    sorted_e = topk_idx[sort_idx]
    onehot = (topk_idx[:, None] == jnp.arange(N_EXPERTS)[None, :])
    counts = jnp.sum(onehot, axis=0, dtype=jnp.int32)
    ccum = jnp.concatenate([jnp.zeros(1, jnp.int32), jnp.cumsum(counts)[:-1]])
    padded = ((counts + BT - 1) // BT) * BT
    offs = jnp.concatenate([jnp.zeros(1, jnp.int32), jnp.cumsum(padded)[:-1]])
    rank = jnp.arange(T, dtype=jnp.int32) - ccum[sorted_e]
    padded_pos = offs[sorted_e] + rank
    g = jnp.zeros(T_PAD, jnp.int32).at[padded_pos].set(sort_idx)
    probs_sorted = jnp.zeros((T_PAD,), jnp.float32).at[padded_pos].set(
        topk_prob[sort_idx])
    pos_token = jnp.zeros(T, jnp.int32).at[sort_idx].set(padded_pos)
    starts = jnp.arange(NB, dtype=jnp.int32) * BT
    total = jnp.sum(padded)
    block_e = jnp.clip(jnp.searchsorted(offs, starts, side='right') - 1,
                       0, N_EXPERTS - 1).astype(jnp.int32)
    block_valid = (starts < total).astype(jnp.int32)

    h_sorted = jnp.take(h, g, axis=0)

    routed_sorted = _moe_ffn(h_sorted, ew1q, ew2q, probs_sorted[:, None],
                             block_e, block_valid)

    shared_out = _shared_ffn(h, sw1q, sw2q)

    routed = jnp.take(routed_sorted, pos_token, axis=0)
    return (shared_out + routed).reshape(x.shape)


# speedup vs baseline: 1.2760x; 1.1094x over previous
import functools

import jax
import jax.numpy as jnp
from jax.experimental import pallas as pl
from jax.experimental.pallas import tpu as pltpu

D_MODEL = 1024
EXPERT_DIM = 2048
N_EXPERTS = 8
T = 2 * 2048
BT = 256
NB = (T + N_EXPERTS * BT) // BT
T_PAD = NB * BT


def _quant_body(w_ref, o_ref):
    w = w_ref[...]
    scale = jnp.clip(jnp.mean(jnp.abs(w), axis=1, keepdims=True), 1e-05, None)
    wq = jnp.clip(jnp.round(w / scale), -1.0, 1.0)
    o_ref[...] = wq * scale


def _quantize(w):
    r, c = w.shape
    rb = min(r, 512)
    return pl.pallas_call(
        _quant_body,
        grid=(r // rb,),
        in_specs=[pl.BlockSpec((rb, c), lambda i: (i, 0))],
        out_specs=pl.BlockSpec((rb, c), lambda i: (i, 0)),
        out_shape=jax.ShapeDtypeStruct((r, c), jnp.float32),
    )(w)


def _ln_router_body(x_ref, g_ref, b_ref, rw_ref, h_ref, p_ref, i_ref):
    x = x_ref[...]
    mu = jnp.mean(x, axis=1, keepdims=True)
    var = jnp.mean(jnp.square(x - mu), axis=1, keepdims=True)
    h = (x - mu) / jnp.sqrt(var + 1e-05) * g_ref[...] + b_ref[...]
    h_ref[...] = h
    logits = jax.lax.dot_general(h, rw_ref[...], (((1,), (1,)), ((), ())),
                                 preferred_element_type=jnp.float32)
    m = jnp.max(logits, axis=1, keepdims=True)
    e = jnp.exp(logits - m)
    p = e / jnp.sum(e, axis=1, keepdims=True)
    pm = jnp.max(p, axis=1, keepdims=True)
    lane = jax.lax.broadcasted_iota(jnp.int32, p.shape, 1)
    idx = jnp.min(jnp.where(p == pm, lane, N_EXPERTS), axis=1, keepdims=True)
    p_ref[...] = pm
    i_ref[...] = idx


def _ln_router(x2d, gamma, beta, router_w):
    return pl.pallas_call(
        _ln_router_body,
        grid=(T // BT,),
        in_specs=[
            pl.BlockSpec((BT, D_MODEL), lambda i: (i, 0)),
            pl.BlockSpec((1, D_MODEL), lambda i: (0, 0)),
            pl.BlockSpec((1, D_MODEL), lambda i: (0, 0)),
            pl.BlockSpec((N_EXPERTS, D_MODEL), lambda i: (0, 0)),
        ],
        out_specs=[
            pl.BlockSpec((BT, D_MODEL), lambda i: (i, 0)),
            pl.BlockSpec((BT, 1), lambda i: (i, 0)),
            pl.BlockSpec((BT, 1), lambda i: (i, 0)),
        ],
        out_shape=[
            jax.ShapeDtypeStruct((T, D_MODEL), jnp.float32),
            jax.ShapeDtypeStruct((T, 1), jnp.float32),
            jax.ShapeDtypeStruct((T, 1), jnp.int32),
        ],
    )(x2d, gamma.reshape(1, -1), beta.reshape(1, -1), router_w)


def _shared_ffn_body(h_ref, w1_ref, w2_ref, o_ref):
    a = jax.lax.dot_general(h_ref[...], w1_ref[...], (((1,), (1,)), ((), ())),
                            preferred_element_type=jnp.float32)
    a = a * jax.lax.logistic(a)
    o_ref[...] = jax.lax.dot_general(a, w2_ref[...], (((1,), (1,)), ((), ())),
                                     preferred_element_type=jnp.float32)


def _shared_ffn(h, w1q, w2q):
    return pl.pallas_call(
        _shared_ffn_body,
        grid=(T // BT,),
        in_specs=[
            pl.BlockSpec((BT, D_MODEL), lambda i: (i, 0)),
            pl.BlockSpec((EXPERT_DIM, D_MODEL), lambda i: (0, 0)),
            pl.BlockSpec((D_MODEL, EXPERT_DIM), lambda i: (0, 0)),
        ],
        out_specs=pl.BlockSpec((BT, D_MODEL), lambda i: (i, 0)),
        out_shape=jax.ShapeDtypeStruct((T, D_MODEL), jnp.float32),
    )(h, w1q, w2q)


def _moe_ffn_body(be_ref, valid_ref, hs_ref, w1_ref, w2_ref, ps_ref, o_ref):
    i = pl.program_id(0)

    @pl.when(valid_ref[i] != 0)
    def _():
        a = jax.lax.dot_general(hs_ref[...], w1_ref[0], (((1,), (1,)), ((), ())),
                                preferred_element_type=jnp.float32)
        a = a * jax.lax.logistic(a)
        o = jax.lax.dot_general(a, w2_ref[0], (((1,), (1,)), ((), ())),
                                preferred_element_type=jnp.float32)
        o_ref[...] = o * ps_ref[...]


def _moe_ffn(h_sorted, ew1q, ew2q, probs_sorted, block_e, block_valid):
    grid_spec = pltpu.PrefetchScalarGridSpec(
        num_scalar_prefetch=2,
        grid=(NB,),
        in_specs=[
            pl.BlockSpec((BT, D_MODEL), lambda i, be, v: (i, 0)),
            pl.BlockSpec((1, EXPERT_DIM, D_MODEL), lambda i, be, v: (be[i], 0, 0)),
            pl.BlockSpec((1, D_MODEL, EXPERT_DIM), lambda i, be, v: (be[i], 0, 0)),
            pl.BlockSpec((BT, 1), lambda i, be, v: (i, 0)),
        ],
        out_specs=pl.BlockSpec((BT, D_MODEL), lambda i, be, v: (i, 0)),
    )
    return pl.pallas_call(
        _moe_ffn_body,
        grid_spec=grid_spec,
        out_shape=jax.ShapeDtypeStruct((T_PAD, D_MODEL), jnp.float32),
    )(block_e, block_valid, h_sorted, ew1q, ew2q, probs_sorted)


def kernel(x, gamma, beta, shared_w1, shared_w2, experts_w1, experts_w2, router_w):
    x2d = x.reshape(T, D_MODEL)

    sw1q = _quantize(shared_w1)
    sw2q = _quantize(shared_w2)
    ew1q = _quantize(experts_w1.reshape(N_EXPERTS * EXPERT_DIM, D_MODEL)).reshape(
        N_EXPERTS, EXPERT_DIM, D_MODEL)
    ew2q = _quantize(experts_w2.reshape(N_EXPERTS * D_MODEL, EXPERT_DIM)).reshape(
        N_EXPERTS, D_MODEL, EXPERT_DIM)

    h, topk_prob, topk_idx = _ln_router(x2d, gamma, beta, router_w)
    topk_prob = topk_prob[:, 0]
    topk_idx = topk_idx[:, 0]

    sort_idx = jnp.argsort(topk_idx)
    sorted_e = topk_idx[sort_idx]
    onehot = (topk_idx[:, None] == jnp.arange(N_EXPERTS)[None, :])
    counts = jnp.sum(onehot, axis=0, dtype=jnp.int32)
    ccum = jnp.concatenate([jnp.zeros(1, jnp.int32), jnp.cumsum(counts)[:-1]])
    padded = ((counts + BT - 1) // BT) * BT
    offs = jnp.concatenate([jnp.zeros(1, jnp.int32), jnp.cumsum(padded)[:-1]])
    rank = jnp.arange(T, dtype=jnp.int32) - ccum[sorted_e]
    padded_pos = offs[sorted_e] + rank
    g = jnp.zeros(T_PAD, jnp.int32).at[padded_pos].set(sort_idx)
    probs_sorted = jnp.zeros((T_PAD,), jnp.float32).at[padded_pos].set(
        topk_prob[sort_idx])
    pos_token = jnp.zeros(T, jnp.int32).at[sort_idx].set(padded_pos)
    starts = jnp.arange(NB, dtype=jnp.int32) * BT
    total = jnp.sum(padded)
    block_e = jnp.clip(jnp.searchsorted(offs, starts, side='right') - 1,
                       0, N_EXPERTS - 1).astype(jnp.int32)
    block_valid = (starts < total).astype(jnp.int32)

    h_sorted = jnp.take(h.astype(jnp.bfloat16), g, axis=0).astype(jnp.float32)

    routed_sorted = _moe_ffn(h_sorted, ew1q, ew2q, probs_sorted[:, None],
                             block_e, block_valid)

    shared_out = _shared_ffn(h, sw1q, sw2q)

    routed = jnp.take(routed_sorted.astype(jnp.bfloat16), pos_token,
                      axis=0).astype(jnp.float32)
    return (shared_out + routed).reshape(x.shape)
